# initial kernel scaffold (unmeasured)
import jax
import jax.numpy as jnp
from jax import lax
from jax.experimental import pallas as pl
from jax.experimental.pallas import tpu as pltpu

N_DEV = 4


def kernel(x, w_mat):
    x = x.astype(jnp.bfloat16)
    w = w_mat.astype(jnp.bfloat16)
    m_per, k = x.shape
    _, n_loc = w.shape

    def body(x_ref, w_ref, out_ref, comm_ref, send_sems, recv_sems,
             amax_ref, amax_send_sems, amax_recv_sems):
        my = lax.axis_index("i")
        left = lax.rem(my + N_DEV - 1, N_DEV)
        right = lax.rem(my + 1, N_DEV)

        barrier = pltpu.get_barrier_semaphore()
        for d in range(1, N_DEV):
            peer = lax.rem(my + d, N_DEV)
            pl.semaphore_signal(barrier, inc=1, device_id=(peer,),
                                device_id_type=pl.DeviceIdType.MESH)
        pl.semaphore_wait(barrier, N_DEV - 1)

        def gemm_chunk(chunk, origin):
            y = jnp.dot(chunk, w_ref[...],
                        preferred_element_type=jnp.float32)
            out_ref[pl.ds(origin * m_per, m_per), :] = y
            return jnp.max(jnp.abs(y))

        m_run = jnp.float32(0.0)
        for h in range(N_DEV - 1):
            send_slot = h % 2
            recv_slot = (h + 1) % 2
            src = x_ref.at[...] if h == 0 else comm_ref.at[send_slot]
            rdma = pltpu.make_async_remote_copy(
                src_ref=src,
                dst_ref=comm_ref.at[recv_slot],
                send_sem=send_sems.at[send_slot],
                recv_sem=recv_sems.at[recv_slot],
                device_id=(right,),
                device_id_type=pl.DeviceIdType.MESH,
            )
            rdma.start()
            chunk = x_ref[...] if h == 0 else comm_ref[send_slot]
            m_run = jnp.maximum(m_run, gemm_chunk(chunk, lax.rem(my + N_DEV - h, N_DEV) if h else my))
            rdma.wait()
        m_run = jnp.maximum(
            m_run,
            gemm_chunk(comm_ref[(N_DEV - 1) % 2], lax.rem(my + 1, N_DEV)))

        amax_ref[pl.ds(my, 1), :] = jnp.full((1, 128), m_run, jnp.float32)
        sends = []
        for d in range(1, N_DEV):
            peer = lax.rem(my + d, N_DEV)
            c = pltpu.make_async_remote_copy(
                src_ref=amax_ref.at[pl.ds(my, 1)],
                dst_ref=amax_ref.at[pl.ds(my, 1)],
                send_sem=amax_send_sems.at[d],
                recv_sem=amax_recv_sems.at[d],
                device_id=(peer,),
                device_id_type=pl.DeviceIdType.MESH,
            )
            c.start()
            sends.append(c)
        for d in range(1, N_DEV):
            src_pos = lax.rem(my + N_DEV - d, N_DEV)
            r = pltpu.make_async_remote_copy(
                src_ref=amax_ref.at[pl.ds(my, 1)],
                dst_ref=amax_ref.at[pl.ds(src_pos, 1)],
                send_sem=amax_send_sems.at[d],
                recv_sem=amax_recv_sems.at[d],
                device_id=(src_pos,),
                device_id_type=pl.DeviceIdType.MESH,
            )
            r.wait_recv()
        for c in sends:
            c.wait_send()

        g = jnp.max(amax_ref[...])
        scale = g / 127.0
        rows_per_blk = 512
        for b in range((N_DEV * m_per) // rows_per_blk):
            blk = out_ref[b * rows_per_blk:(b + 1) * rows_per_blk, :]
            q = jnp.clip(jnp.round(blk / scale), -127.0, 127.0)
            out_ref[b * rows_per_blk:(b + 1) * rows_per_blk, :] = q * scale

    out_shape = jax.ShapeDtypeStruct((N_DEV * m_per, n_loc), jnp.float32)
    return pl.pallas_call(
        body,
        out_shape=out_shape,
        in_specs=[pl.BlockSpec(memory_space=pltpu.VMEM),
                  pl.BlockSpec(memory_space=pltpu.VMEM)],
        out_specs=pl.BlockSpec(memory_space=pltpu.VMEM),
        scratch_shapes=[
            pltpu.VMEM((2, m_per, k), jnp.bfloat16),
            pltpu.SemaphoreType.DMA((2,)),
            pltpu.SemaphoreType.DMA((2,)),
            pltpu.VMEM((N_DEV, 128), jnp.float32),
            pltpu.SemaphoreType.DMA((N_DEV,)),
            pltpu.SemaphoreType.DMA((N_DEV,)),
        ],
        compiler_params=pltpu.CompilerParams(
            collective_id=0,
            vmem_limit_bytes=128 * 1024 * 1024,
        ),
    )(x, w)


# baseline (device time: 397532 ns/iter reference)
import jax
import jax.numpy as jnp
from jax import lax
from jax.experimental import pallas as pl
from jax.experimental.pallas import tpu as pltpu

N_DEV = 4
MB = 512


def kernel(x, w_mat):
    x = x.astype(jnp.bfloat16)
    w = w_mat.astype(jnp.bfloat16)
    m_per, k = x.shape
    _, n_loc = w.shape
    m_tot = N_DEV * m_per

    def body(x_ref, w_ref, out_ref, comm_ref, send_sems, recv_sems,
             stage_ref, stage_sems,
             amax_ref, amax_send_sems, amax_recv_sems):
        my = lax.axis_index("i")
        right = lax.rem(my + 1, N_DEV)

        barrier = pltpu.get_barrier_semaphore()
        for d in range(1, N_DEV):
            peer = lax.rem(my + d, N_DEV)
            pl.semaphore_signal(barrier, inc=1, device_id=(peer,),
                                device_id_type=pl.DeviceIdType.MESH)
        pl.semaphore_wait(barrier, N_DEV - 1)

        pending = [None, None]
        slot_ctr = [0]

        def gemm_chunk(m_run, origin, h):
            for mb in range(m_per // MB):
                slot = slot_ctr[0] % 2
                slot_ctr[0] += 1
                if pending[slot] is not None:
                    pending[slot].wait()
                rows = (x_ref[mb * MB:(mb + 1) * MB, :] if h == 0
                        else comm_ref[h % 2, mb * MB:(mb + 1) * MB, :])
                y = jnp.dot(rows, w_ref[...],
                            preferred_element_type=jnp.float32)
                stage_ref[slot] = y
                m_run = jnp.maximum(m_run, jnp.max(jnp.abs(y)))
                cp = pltpu.make_async_copy(
                    stage_ref.at[slot],
                    out_ref.at[pl.ds(origin * m_per + mb * MB, MB), :],
                    stage_sems.at[slot])
                cp.start()
                pending[slot] = cp
            return m_run

        m_run = jnp.float32(0.0)
        for h in range(N_DEV - 1):
            send_slot = h % 2
            recv_slot = (h + 1) % 2
            rdma = pltpu.make_async_remote_copy(
                src_ref=x_ref if h == 0 else comm_ref.at[send_slot],
                dst_ref=comm_ref.at[recv_slot],
                send_sem=send_sems.at[send_slot],
                recv_sem=recv_sems.at[recv_slot],
                device_id=(right,),
                device_id_type=pl.DeviceIdType.MESH,
            )
            rdma.start()
            origin = lax.rem(my + N_DEV - h, N_DEV) if h else my
            m_run = gemm_chunk(m_run, origin, h)
            rdma.wait()
        m_run = gemm_chunk(m_run, lax.rem(my + 1, N_DEV), N_DEV - 1)
        for cp in pending:
            if cp is not None:
                cp.wait()

        amax_ref[pl.ds(my, 1), :] = jnp.full((1, 128), m_run, jnp.float32)
        sends = []
        for d in range(1, N_DEV):
            peer = lax.rem(my + d, N_DEV)
            c = pltpu.make_async_remote_copy(
                src_ref=amax_ref.at[pl.ds(my, 1)],
                dst_ref=amax_ref.at[pl.ds(my, 1)],
                send_sem=amax_send_sems.at[d],
                recv_sem=amax_recv_sems.at[d],
                device_id=(peer,),
                device_id_type=pl.DeviceIdType.MESH,
            )
            c.start()
            sends.append(c)
        for d in range(1, N_DEV):
            src_pos = lax.rem(my + N_DEV - d, N_DEV)
            r = pltpu.make_async_remote_copy(
                src_ref=amax_ref.at[pl.ds(my, 1)],
                dst_ref=amax_ref.at[pl.ds(src_pos, 1)],
                send_sem=amax_send_sems.at[d],
                recv_sem=amax_recv_sems.at[d],
                device_id=(src_pos,),
                device_id_type=pl.DeviceIdType.MESH,
            )
            r.wait_recv()
        for c in sends:
            c.wait_send()

        g = jnp.max(amax_ref[...])
        scale = g / 127.0
        for b in range(m_tot // MB):
            slot = b % 2
            ld = pltpu.make_async_copy(
                out_ref.at[pl.ds(b * MB, MB), :],
                stage_ref.at[slot], stage_sems.at[slot])
            ld.start()
            ld.wait()
            blk = stage_ref[slot]
            q = jnp.clip(jnp.round(blk / scale), -127.0, 127.0)
            stage_ref[slot] = q * scale
            st = pltpu.make_async_copy(
                stage_ref.at[slot],
                out_ref.at[pl.ds(b * MB, MB), :], stage_sems.at[slot])
            st.start()
            st.wait()

    out_shape = jax.ShapeDtypeStruct((m_tot, n_loc), jnp.float32)
    return pl.pallas_call(
        body,
        out_shape=out_shape,
        in_specs=[pl.BlockSpec(memory_space=pltpu.VMEM),
                  pl.BlockSpec(memory_space=pltpu.VMEM)],
        out_specs=pl.BlockSpec(memory_space=pl.ANY),
        scratch_shapes=[
            pltpu.VMEM((2, m_per, k), jnp.bfloat16),
            pltpu.SemaphoreType.DMA((2,)),
            pltpu.SemaphoreType.DMA((2,)),
            pltpu.VMEM((2, MB, n_loc), jnp.float32),
            pltpu.SemaphoreType.DMA((2,)),
            pltpu.VMEM((N_DEV, 128), jnp.float32),
            pltpu.SemaphoreType.DMA((N_DEV,)),
            pltpu.SemaphoreType.DMA((N_DEV,)),
        ],
        compiler_params=pltpu.CompilerParams(
            collective_id=0,
            vmem_limit_bytes=100 * 1024 * 1024,
        ),
    )(x, w)


# device time: 251648 ns/iter; 1.5797x vs baseline; 1.5797x over previous
import jax
import jax.numpy as jnp
from jax import lax
from jax.experimental import pallas as pl
from jax.experimental.pallas import tpu as pltpu

N_DEV = 4
MB = 512
SLOT_L, SLOT_R, SLOT_O = 0, 1, 2


def kernel(x, w_mat):
    x = x.astype(jnp.bfloat16)
    w = w_mat.astype(jnp.bfloat16)
    m_per, k = x.shape
    _, n_loc = w.shape
    m_tot = N_DEV * m_per
    half = m_per // 2

    def body(x_ref, w_ref, out_ref, comm_ref,
             p1_send, p1_recv, p2_send, p2_recv,
             stage_ref, ld_sems, st_sems,
             amax_ref, amax_send_sems, amax_recv_sems):
        my = lax.axis_index("i")
        left = lax.rem(my + N_DEV - 1, N_DEV)
        right = lax.rem(my + 1, N_DEV)

        barrier = pltpu.get_barrier_semaphore()
        for d in range(1, N_DEV):
            peer = lax.rem(my + d, N_DEV)
            pl.semaphore_signal(barrier, inc=1, device_id=(peer,),
                                device_id_type=pl.DeviceIdType.MESH)
        pl.semaphore_wait(barrier, N_DEV - 1)

        pending = [None, None]
        slot_ctr = [0]

        def gemm_chunk(m_run, origin, row_src):
            for mb in range(m_per // MB):
                slot = slot_ctr[0] % 2
                slot_ctr[0] += 1
                if pending[slot] is not None:
                    pending[slot].wait()
                y = jnp.dot(row_src(mb), w_ref[...],
                            preferred_element_type=jnp.float32)
                stage_ref[slot] = y
                m_run = jnp.maximum(m_run, jnp.max(jnp.abs(y)))
                cp = pltpu.make_async_copy(
                    stage_ref.at[slot],
                    out_ref.at[pl.ds(origin * m_per + mb * MB, MB), :],
                    st_sems.at[slot])
                cp.start()
                pending[slot] = cp
            return m_run

        p1r = pltpu.make_async_remote_copy(
            src_ref=x_ref, dst_ref=comm_ref.at[SLOT_L],
            send_sem=p1_send.at[0], recv_sem=p1_recv.at[SLOT_L],
            device_id=(right,), device_id_type=pl.DeviceIdType.MESH)
        p1r.start()
        p1l = pltpu.make_async_remote_copy(
            src_ref=x_ref, dst_ref=comm_ref.at[SLOT_R],
            send_sem=p1_send.at[1], recv_sem=p1_recv.at[SLOT_R],
            device_id=(left,), device_id_type=pl.DeviceIdType.MESH)
        p1l.start()

        m_run = jnp.float32(0.0)
        m_run = gemm_chunk(m_run, my,
                           lambda mb: x_ref[mb * MB:(mb + 1) * MB, :])

        p1r.wait_recv()
        p2r = pltpu.make_async_remote_copy(
            src_ref=comm_ref.at[SLOT_L, pl.ds(0, half), :],
            dst_ref=comm_ref.at[SLOT_O, pl.ds(0, half), :],
            send_sem=p2_send.at[0], recv_sem=p2_recv.at[0],
            device_id=(right,), device_id_type=pl.DeviceIdType.MESH)
        p2r.start()
        p1l.wait_recv()
        p2l = pltpu.make_async_remote_copy(
            src_ref=comm_ref.at[SLOT_R, pl.ds(half, half), :],
            dst_ref=comm_ref.at[SLOT_O, pl.ds(half, half), :],
            send_sem=p2_send.at[1], recv_sem=p2_recv.at[1],
            device_id=(left,), device_id_type=pl.DeviceIdType.MESH)
        p2l.start()

        m_run = gemm_chunk(
            m_run, lax.rem(my + N_DEV - 1, N_DEV),
            lambda mb: comm_ref[SLOT_L, mb * MB:(mb + 1) * MB, :])
        m_run = gemm_chunk(
            m_run, lax.rem(my + 1, N_DEV),
            lambda mb: comm_ref[SLOT_R, mb * MB:(mb + 1) * MB, :])

        p2r.wait_recv()
        p2l.wait_recv()
        m_run = gemm_chunk(
            m_run, lax.rem(my + 2, N_DEV),
            lambda mb: comm_ref[SLOT_O, mb * MB:(mb + 1) * MB, :])

        for cp in pending:
            if cp is not None:
                cp.wait()
        for c in (p1r, p1l, p2r, p2l):
            c.wait_send()

        amax_ref[pl.ds(my, 1), :] = jnp.full((1, 128), m_run, jnp.float32)
        sends = []
        for d in range(1, N_DEV):
            peer = lax.rem(my + d, N_DEV)
            c = pltpu.make_async_remote_copy(
                src_ref=amax_ref.at[pl.ds(my, 1)],
                dst_ref=amax_ref.at[pl.ds(my, 1)],
                send_sem=amax_send_sems.at[d],
                recv_sem=amax_recv_sems.at[d],
                device_id=(peer,),
                device_id_type=pl.DeviceIdType.MESH,
            )
            c.start()
            sends.append(c)
        for d in range(1, N_DEV):
            src_pos = lax.rem(my + N_DEV - d, N_DEV)
            r = pltpu.make_async_remote_copy(
                src_ref=amax_ref.at[pl.ds(my, 1)],
                dst_ref=amax_ref.at[pl.ds(src_pos, 1)],
                send_sem=amax_send_sems.at[d],
                recv_sem=amax_recv_sems.at[d],
                device_id=(src_pos,),
                device_id_type=pl.DeviceIdType.MESH,
            )
            r.wait_recv()
        for c in sends:
            c.wait_send()

        g = jnp.max(amax_ref[...])
        scale = g / 127.0
        n_blk = m_tot // MB

        def load(b):
            c = pltpu.make_async_copy(
                out_ref.at[pl.ds(b * MB, MB), :],
                stage_ref.at[b % 2], ld_sems.at[b % 2])
            c.start()
            return c

        ld_pend = [load(0), None]
        st_pend = [None, None]
        for b in range(n_blk):
            slot = b % 2
            other = (b + 1) % 2
            ld_pend[slot].wait()
            if b + 1 < n_blk:
                if st_pend[other] is not None:
                    st_pend[other].wait()
                ld_pend[other] = load(b + 1)
            blk = stage_ref[slot]
            q = jnp.clip(jnp.round(blk / scale), -127.0, 127.0)
            stage_ref[slot] = q * scale
            st = pltpu.make_async_copy(
                stage_ref.at[slot],
                out_ref.at[pl.ds(b * MB, MB), :], st_sems.at[slot])
            st.start()
            st_pend[slot] = st
        for c in st_pend:
            if c is not None:
                c.wait()

    out_shape = jax.ShapeDtypeStruct((m_tot, n_loc), jnp.float32)
    return pl.pallas_call(
        body,
        out_shape=out_shape,
        in_specs=[pl.BlockSpec(memory_space=pltpu.VMEM),
                  pl.BlockSpec(memory_space=pltpu.VMEM)],
        out_specs=pl.BlockSpec(memory_space=pl.ANY),
        scratch_shapes=[
            pltpu.VMEM((3, m_per, k), jnp.bfloat16),
            pltpu.SemaphoreType.DMA((2,)),
            pltpu.SemaphoreType.DMA((2,)),
            pltpu.SemaphoreType.DMA((2,)),
            pltpu.SemaphoreType.DMA((2,)),
            pltpu.VMEM((2, MB, n_loc), jnp.float32),
            pltpu.SemaphoreType.DMA((2,)),
            pltpu.SemaphoreType.DMA((2,)),
            pltpu.VMEM((N_DEV, 128), jnp.float32),
            pltpu.SemaphoreType.DMA((N_DEV,)),
            pltpu.SemaphoreType.DMA((N_DEV,)),
        ],
        compiler_params=pltpu.CompilerParams(
            collective_id=0,
            vmem_limit_bytes=100 * 1024 * 1024,
        ),
    )(x, w)


# device time: 249781 ns/iter; 1.5915x vs baseline; 1.0075x over previous
import jax
import jax.numpy as jnp
from jax import lax
from jax.experimental import pallas as pl
from jax.experimental.pallas import tpu as pltpu

N_DEV = 4
MB = 512
SLOT_L, SLOT_R, SLOT_O = 0, 1, 2


def kernel(x, w_mat):
    x = x.astype(jnp.bfloat16)
    w = w_mat.astype(jnp.bfloat16)
    m_per, k = x.shape
    _, n_loc = w.shape
    m_tot = N_DEV * m_per

    def body(x_ref, w_bf, out_ref, comm_ref,
             p1_send, p1_recv, p2_send, p2_recv,
             stage_ref, ld_sems, st_sems,
             amax_ref, amax_send_sems, amax_recv_sems):
        my = lax.axis_index("i")
        left = lax.rem(my + N_DEV - 1, N_DEV)
        right = lax.rem(my + 1, N_DEV)

        barrier = pltpu.get_barrier_semaphore()
        for d in range(1, N_DEV):
            peer = lax.rem(my + d, N_DEV)
            pl.semaphore_signal(barrier, inc=1, device_id=(peer,),
                                device_id_type=pl.DeviceIdType.MESH)
        pl.semaphore_wait(barrier, N_DEV - 1)

        p1r = pltpu.make_async_remote_copy(
            src_ref=x_ref, dst_ref=comm_ref.at[SLOT_L],
            send_sem=p1_send.at[0], recv_sem=p1_recv.at[SLOT_L],
            device_id=(right,), device_id_type=pl.DeviceIdType.MESH)
        p1r.start()
        p1l = pltpu.make_async_remote_copy(
            src_ref=x_ref, dst_ref=comm_ref.at[SLOT_R],
            send_sem=p1_send.at[1], recv_sem=p1_recv.at[SLOT_R],
            device_id=(left,), device_id_type=pl.DeviceIdType.MESH)
        p1l.start()

        pending = [None, None]
        slot_ctr = [0]

        def gemm_block(m_run, out_row0, rows):
            slot = slot_ctr[0] % 2
            slot_ctr[0] += 1
            if pending[slot] is not None:
                pending[slot].wait()
            y = jnp.dot(rows, w_bf[...], preferred_element_type=jnp.float32)
            stage_ref[slot] = y
            m_run = jnp.maximum(m_run, jnp.max(jnp.abs(y)))
            cp = pltpu.make_async_copy(
                stage_ref.at[slot],
                out_ref.at[pl.ds(out_row0, MB), :],
                st_sems.at[slot])
            cp.start()
            pending[slot] = cp
            return m_run

        def gemm_chunk(m_run, origin, row_fn):
            for mb in range(m_per // MB):
                m_run = gemm_block(m_run, origin * m_per + mb * MB,
                                   row_fn(mb))
            return m_run

        m_run = jnp.float32(0.0)
        m_run = gemm_chunk(m_run, my,
                           lambda mb: x_ref[mb * MB:(mb + 1) * MB, :])

        hf = m_per // 2
        p1r.wait_recv()
        p2r = pltpu.make_async_remote_copy(
            src_ref=comm_ref.at[SLOT_L, pl.ds(0, hf), :],
            dst_ref=comm_ref.at[SLOT_O, pl.ds(0, hf), :],
            send_sem=p2_send.at[0], recv_sem=p2_recv.at[0],
            device_id=(right,), device_id_type=pl.DeviceIdType.MESH)
        p2r.start()
        p1l.wait_recv()
        p2l = pltpu.make_async_remote_copy(
            src_ref=comm_ref.at[SLOT_R, pl.ds(hf, hf), :],
            dst_ref=comm_ref.at[SLOT_O, pl.ds(hf, hf), :],
            send_sem=p2_send.at[1], recv_sem=p2_recv.at[1],
            device_id=(left,), device_id_type=pl.DeviceIdType.MESH)
        p2l.start()

        m_run = gemm_chunk(
            m_run, lax.rem(my + N_DEV - 1, N_DEV),
            lambda mb: comm_ref[SLOT_L, mb * MB:(mb + 1) * MB, :])
        m_run = gemm_chunk(
            m_run, lax.rem(my + 1, N_DEV),
            lambda mb: comm_ref[SLOT_R, mb * MB:(mb + 1) * MB, :])

        diag = lax.rem(my + 2, N_DEV)
        half = m_per // 2
        p2r.wait_recv()
        for mb in range(half // MB):
            m_run = gemm_block(m_run, diag * m_per + mb * MB,
                               comm_ref[SLOT_O, mb * MB:(mb + 1) * MB, :])
        p2l.wait_recv()
        for mb in range(half // MB, m_per // MB):
            m_run = gemm_block(m_run, diag * m_per + mb * MB,
                               comm_ref[SLOT_O, mb * MB:(mb + 1) * MB, :])

        for cp in pending:
            if cp is not None:
                cp.wait()

        n_blk = m_tot // MB

        def qload(b):
            c = pltpu.make_async_copy(
                out_ref.at[pl.ds(b * MB, MB), :],
                stage_ref.at[b % 2], ld_sems.at[b % 2])
            c.start()
            return c

        ld_pend = [qload(0), None]

        for c in (p1r, p1l, p2r, p2l):
            c.wait_send()

        amax_ref[pl.ds(my, 1), :] = jnp.full((1, 128), m_run, jnp.float32)
        sends = []
        for d in range(1, N_DEV):
            peer = lax.rem(my + d, N_DEV)
            c = pltpu.make_async_remote_copy(
                src_ref=amax_ref.at[pl.ds(my, 1)],
                dst_ref=amax_ref.at[pl.ds(my, 1)],
                send_sem=amax_send_sems.at[d],
                recv_sem=amax_recv_sems.at[d],
                device_id=(peer,),
                device_id_type=pl.DeviceIdType.MESH,
            )
            c.start()
            sends.append(c)
        for d in range(1, N_DEV):
            src_pos = lax.rem(my + N_DEV - d, N_DEV)
            r = pltpu.make_async_remote_copy(
                src_ref=amax_ref.at[pl.ds(my, 1)],
                dst_ref=amax_ref.at[pl.ds(src_pos, 1)],
                send_sem=amax_send_sems.at[d],
                recv_sem=amax_recv_sems.at[d],
                device_id=(src_pos,),
                device_id_type=pl.DeviceIdType.MESH,
            )
            r.wait_recv()
        for c in sends:
            c.wait_send()

        g = jnp.max(amax_ref[...])
        scale = g / 127.0
        st_pend = [None, None]
        for b in range(n_blk):
            slot = b % 2
            other = (b + 1) % 2
            ld_pend[slot].wait()
            if b + 1 < n_blk:
                if st_pend[other] is not None:
                    st_pend[other].wait()
                ld_pend[other] = qload(b + 1)
            blk = stage_ref[slot]
            q = jnp.clip(jnp.round(blk / scale), -127.0, 127.0)
            stage_ref[slot] = q * scale
            st = pltpu.make_async_copy(
                stage_ref.at[slot],
                out_ref.at[pl.ds(b * MB, MB), :], st_sems.at[slot])
            st.start()
            st_pend[slot] = st
        for c in st_pend:
            if c is not None:
                c.wait()

    out_shape = jax.ShapeDtypeStruct((m_tot, n_loc), jnp.float32)
    return pl.pallas_call(
        body,
        out_shape=out_shape,
        in_specs=[pl.BlockSpec(memory_space=pltpu.VMEM),
                  pl.BlockSpec(memory_space=pltpu.VMEM)],
        out_specs=pl.BlockSpec(memory_space=pl.ANY),
        scratch_shapes=[
            pltpu.VMEM((3, m_per, k), jnp.bfloat16),
            pltpu.SemaphoreType.DMA((2,)),
            pltpu.SemaphoreType.DMA((2,)),
            pltpu.SemaphoreType.DMA((2,)),
            pltpu.SemaphoreType.DMA((2,)),
            pltpu.VMEM((2, MB, n_loc), jnp.float32),
            pltpu.SemaphoreType.DMA((2,)),
            pltpu.SemaphoreType.DMA((2,)),
            pltpu.VMEM((N_DEV, 128), jnp.float32),
            pltpu.SemaphoreType.DMA((N_DEV,)),
            pltpu.SemaphoreType.DMA((N_DEV,)),
        ],
        compiler_params=pltpu.CompilerParams(
            collective_id=0,
            vmem_limit_bytes=100 * 1024 * 1024,
        ),
    )(x, w)


# device time: 249746 ns/iter; 1.5917x vs baseline; 1.0001x over previous
import jax
import jax.numpy as jnp
from jax import lax
from jax.experimental import pallas as pl
from jax.experimental.pallas import tpu as pltpu

N_DEV = 4
MB = 512
SLOT_L, SLOT_R, SLOT_O = 0, 1, 2


def kernel(x, w_mat):
    x = x.astype(jnp.bfloat16)
    w = w_mat.astype(jnp.bfloat16)
    m_per, k = x.shape
    _, n_loc = w.shape
    m_tot = N_DEV * m_per

    def body(x_ref, w_bf, out_ref, comm_ref,
             p1_send, p1_recv, p2_send, p2_recv,
             stage_ref, ld_sems, st_sems,
             amax_ref, amax_send_sems, amax_recv_sems):
        my = lax.axis_index("i")
        left = lax.rem(my + N_DEV - 1, N_DEV)
        right = lax.rem(my + 1, N_DEV)

        barrier = pltpu.get_barrier_semaphore()
        for d in range(1, N_DEV):
            peer = lax.rem(my + d, N_DEV)
            pl.semaphore_signal(barrier, inc=1, device_id=(peer,),
                                device_id_type=pl.DeviceIdType.MESH)
        pl.semaphore_wait(barrier, N_DEV - 1)

        p1r = pltpu.make_async_remote_copy(
            src_ref=x_ref, dst_ref=comm_ref.at[SLOT_L],
            send_sem=p1_send.at[0], recv_sem=p1_recv.at[SLOT_L],
            device_id=(right,), device_id_type=pl.DeviceIdType.MESH)
        p1r.start()
        p1l = pltpu.make_async_remote_copy(
            src_ref=x_ref, dst_ref=comm_ref.at[SLOT_R],
            send_sem=p1_send.at[1], recv_sem=p1_recv.at[SLOT_R],
            device_id=(left,), device_id_type=pl.DeviceIdType.MESH)
        p1l.start()

        pending = [None, None]
        slot_ctr = [0]

        def gemm_block(m_run, out_row0, rows):
            slot = slot_ctr[0] % 2
            slot_ctr[0] += 1
            if pending[slot] is not None:
                pending[slot].wait()
            for nb in range(2):
                c0, c1 = nb * (n_loc // 2), (nb + 1) * (n_loc // 2)
                stage_ref[slot, :, c0:c1] = jnp.dot(
                    rows, w_bf[:, c0:c1],
                    preferred_element_type=jnp.float32)
                m_run = jnp.maximum(
                    m_run, jnp.max(jnp.abs(stage_ref[slot, :, c0:c1])))
            cp = pltpu.make_async_copy(
                stage_ref.at[slot],
                out_ref.at[pl.ds(out_row0, MB), :],
                st_sems.at[slot])
            cp.start()
            pending[slot] = cp
            return m_run

        def gemm_chunk(m_run, origin, row_fn):
            for mb in range(m_per // MB):
                m_run = gemm_block(m_run, origin * m_per + mb * MB,
                                   row_fn(mb))
            return m_run

        m_run = jnp.float32(0.0)
        m_run = gemm_chunk(m_run, my,
                           lambda mb: x_ref[mb * MB:(mb + 1) * MB, :])

        hf = m_per // 2
        p1r.wait_recv()
        p2r = pltpu.make_async_remote_copy(
            src_ref=comm_ref.at[SLOT_L, pl.ds(0, hf), :],
            dst_ref=comm_ref.at[SLOT_O, pl.ds(0, hf), :],
            send_sem=p2_send.at[0], recv_sem=p2_recv.at[0],
            device_id=(right,), device_id_type=pl.DeviceIdType.MESH)
        p2r.start()
        p1l.wait_recv()
        p2l = pltpu.make_async_remote_copy(
            src_ref=comm_ref.at[SLOT_R, pl.ds(hf, hf), :],
            dst_ref=comm_ref.at[SLOT_O, pl.ds(hf, hf), :],
            send_sem=p2_send.at[1], recv_sem=p2_recv.at[1],
            device_id=(left,), device_id_type=pl.DeviceIdType.MESH)
        p2l.start()

        m_run = gemm_chunk(
            m_run, lax.rem(my + N_DEV - 1, N_DEV),
            lambda mb: comm_ref[SLOT_L, mb * MB:(mb + 1) * MB, :])
        m_run = gemm_chunk(
            m_run, lax.rem(my + 1, N_DEV),
            lambda mb: comm_ref[SLOT_R, mb * MB:(mb + 1) * MB, :])

        diag = lax.rem(my + 2, N_DEV)
        half = m_per // 2
        p2r.wait_recv()
        for mb in range(half // MB):
            m_run = gemm_block(m_run, diag * m_per + mb * MB,
                               comm_ref[SLOT_O, mb * MB:(mb + 1) * MB, :])
        p2l.wait_recv()
        for mb in range(half // MB, m_per // MB):
            m_run = gemm_block(m_run, diag * m_per + mb * MB,
                               comm_ref[SLOT_O, mb * MB:(mb + 1) * MB, :])

        for cp in pending:
            if cp is not None:
                cp.wait()

        n_blk = m_tot // MB

        def qload(b):
            c = pltpu.make_async_copy(
                out_ref.at[pl.ds(b * MB, MB), :],
                stage_ref.at[b % 2], ld_sems.at[b % 2])
            c.start()
            return c

        ld_pend = [qload(0), None]

        for c in (p1r, p1l, p2r, p2l):
            c.wait_send()

        amax_ref[pl.ds(my, 1), :] = jnp.full((1, 128), m_run, jnp.float32)
        sends = []
        for d in range(1, N_DEV):
            peer = lax.rem(my + d, N_DEV)
            c = pltpu.make_async_remote_copy(
                src_ref=amax_ref.at[pl.ds(my, 1)],
                dst_ref=amax_ref.at[pl.ds(my, 1)],
                send_sem=amax_send_sems.at[d],
                recv_sem=amax_recv_sems.at[d],
                device_id=(peer,),
                device_id_type=pl.DeviceIdType.MESH,
            )
            c.start()
            sends.append(c)
        for d in range(1, N_DEV):
            src_pos = lax.rem(my + N_DEV - d, N_DEV)
            r = pltpu.make_async_remote_copy(
                src_ref=amax_ref.at[pl.ds(my, 1)],
                dst_ref=amax_ref.at[pl.ds(src_pos, 1)],
                send_sem=amax_send_sems.at[d],
                recv_sem=amax_recv_sems.at[d],
                device_id=(src_pos,),
                device_id_type=pl.DeviceIdType.MESH,
            )
            r.wait_recv()
        for c in sends:
            c.wait_send()

        g = jnp.max(amax_ref[...])
        scale = g / 127.0
        inv = 127.0 / g
        st_pend = [None, None]
        for b in range(n_blk):
            slot = b % 2
            other = (b + 1) % 2
            ld_pend[slot].wait()
            if b + 1 < n_blk:
                if st_pend[other] is not None:
                    st_pend[other].wait()
                ld_pend[other] = qload(b + 1)
            for sub in range(2):
                r0, r1 = sub * (MB // 2), (sub + 1) * (MB // 2)
                stage_ref[slot, r0:r1, :] = jnp.round(
                    stage_ref[slot, r0:r1, :] * inv) * scale
            st = pltpu.make_async_copy(
                stage_ref.at[slot],
                out_ref.at[pl.ds(b * MB, MB), :], st_sems.at[slot])
            st.start()
            st_pend[slot] = st
        for c in st_pend:
            if c is not None:
                c.wait()

    out_shape = jax.ShapeDtypeStruct((m_tot, n_loc), jnp.float32)
    return pl.pallas_call(
        body,
        out_shape=out_shape,
        in_specs=[pl.BlockSpec(memory_space=pltpu.VMEM),
                  pl.BlockSpec(memory_space=pltpu.VMEM)],
        out_specs=pl.BlockSpec(memory_space=pl.ANY),
        scratch_shapes=[
            pltpu.VMEM((3, m_per, k), jnp.bfloat16),
            pltpu.SemaphoreType.DMA((2,)),
            pltpu.SemaphoreType.DMA((2,)),
            pltpu.SemaphoreType.DMA((2,)),
            pltpu.SemaphoreType.DMA((2,)),
            pltpu.VMEM((2, MB, n_loc), jnp.float32),
            pltpu.SemaphoreType.DMA((2,)),
            pltpu.SemaphoreType.DMA((2,)),
            pltpu.VMEM((N_DEV, 128), jnp.float32),
            pltpu.SemaphoreType.DMA((N_DEV,)),
            pltpu.SemaphoreType.DMA((N_DEV,)),
        ],
        compiler_params=pltpu.CompilerParams(
            collective_id=0,
            vmem_limit_bytes=100 * 1024 * 1024,
        ),
    )(x, w)
